# baseline (speedup 1.0000x reference)
import functools

import jax
import jax.numpy as jnp
from jax import lax
from jax.experimental import pallas as pl
from jax.experimental.pallas import tpu as pltpu
from jax.experimental.pallas import tpu_sc as plsc

_J = 9
_C = 32
_V = _J * _C
_N_NODES = 10000
_N_EDGES = 160000
_RESCALE = 16.0

_BE = 800

_NS = 16
_EPT = _N_EDGES // _NS
_CHUNK = 40
_NCHUNK = _EPT // _CHUNK
_HN = _N_NODES // 2
_TRASH = _HN


def _edge_kernel(xe_ref, w81_ref, env_ref, mask4_ref, W1_ref, b1_ref,
                 lns_ref, lnb_ref, S_ref, W2T_ref, b2T_ref, P_ref,
                 v0_ref, v1_ref, vg2_ref):
    h = jnp.dot(xe_ref[...], W1_ref[...], preferred_element_type=jnp.float32)
    h = h + b1_ref[...]
    mean = jnp.mean(h, axis=-1, keepdims=True)
    var = jnp.mean((h - mean) * (h - mean), axis=-1, keepdims=True)
    h = (h - mean) * jax.lax.rsqrt(var + 1e-5) * lns_ref[...] + lnb_ref[...]
    h = h * jax.nn.sigmoid(h)
    hb = h.astype(jnp.bfloat16)
    env = env_ref[...] * (1.0 / _RESCALE)
    wb = (w81_ref[...] * env).astype(jnp.bfloat16)
    v = None
    for k in range(3):
        a = jnp.dot(wb, S_ref[k], preferred_element_type=jnp.float32)
        bm = jnp.dot(hb, W2T_ref[k], preferred_element_type=jnp.float32)
        bm = bm + b2T_ref[k]
        v = a * bm if v is None else v + a * bm
    v0_ref[...] = v[:, 0:128]
    v1_ref[...] = v[:, 128:256]
    vg232 = v[:, 256:288]
    g = None
    for q in range(4):
        vq = (mask4_ref[:, q:q + 1] * vg232).astype(jnp.bfloat16)
        gq = jnp.dot(vq, P_ref[q], preferred_element_type=jnp.float32)
        g = gq if g is None else g + gq
    vg2_ref[...] = g


def _edge_values(x_edge, wigner81, envelope, mask4, W1, b1, ln_scale,
                 ln_bias, S, W2T, b2T, P):
    n_blocks = _N_EDGES // _BE
    full = lambda shape: pl.BlockSpec(shape, lambda i: (0,) * len(shape))
    return pl.pallas_call(
        _edge_kernel,
        grid=(n_blocks,),
        in_specs=[
            pl.BlockSpec((_BE, 128), lambda i: (i, 0)),
            pl.BlockSpec((_BE, 81), lambda i: (i, 0)),
            pl.BlockSpec((_BE, 1), lambda i: (i, 0)),
            pl.BlockSpec((_BE, 4), lambda i: (i, 0)),
            full((128, 64)),
            full((1, 64)),
            full((1, 64)),
            full((1, 64)),
            full((3, 81, _V)),
            full((3, 64, _V)),
            full((3, 1, _V)),
            full((4, 32, 128)),
        ],
        out_specs=[
            pl.BlockSpec((_BE, 128), lambda i: (i, 0)),
            pl.BlockSpec((_BE, 128), lambda i: (i, 0)),
            pl.BlockSpec((_BE, 128), lambda i: (i, 0)),
        ],
        out_shape=[
            jax.ShapeDtypeStruct((_N_EDGES, 128), jnp.float32),
            jax.ShapeDtypeStruct((_N_EDGES, 128), jnp.float32),
            jax.ShapeDtypeStruct((_N_EDGES, 128), jnp.float32),
        ],
    )(x_edge, wigner81, envelope, mask4, W1, b1, ln_scale, ln_bias,
      S, W2T, b2T, P)


_GROWS = 1256


def _sc_scatter(v0, v1, vg2, x2d, xg2, idx, idxg2):
    mesh = plsc.VectorSubcoreMesh(core_axis_name="c", subcore_axis_name="s")

    @functools.partial(
        pl.kernel,
        mesh=mesh,
        out_type=[
            jax.ShapeDtypeStruct((_N_NODES, 256), jnp.float32),
            jax.ShapeDtypeStruct((2 * _GROWS, 128), jnp.float32),
        ],
        scratch_types=[
            pltpu.VMEM((_CHUNK,), jnp.int32),
            pltpu.VMEM((_CHUNK,), jnp.int32),
            pltpu.VMEM((_CHUNK, 128), jnp.float32),
            pltpu.VMEM((_CHUNK, 128), jnp.float32),
            pltpu.VMEM((_CHUNK,), jnp.int32),
            pltpu.VMEM((_CHUNK,), jnp.int32),
            pltpu.VMEM((_CHUNK, 128), jnp.float32),
            pltpu.VMEM((_CHUNK, 128), jnp.float32),
            pltpu.VMEM_SHARED((_N_NODES, 128), jnp.float32),
            pltpu.VMEM_SHARED((_GROWS, 128), jnp.float32),
            pltpu.SemaphoreType.DMA,
            pltpu.SemaphoreType.DMA,
            pltpu.SemaphoreType.DMA,
            pltpu.SemaphoreType.DMA,
        ],
    )
    def k(v0_hbm, v1_hbm, vg2_hbm, x_hbm, xg2_hbm, idx_hbm, idxg2_hbm,
          om_hbm, og2_hbm, idx_a, idxg2_a, buf_a, bufg2_a,
          idx_b, idxg2_b, buf_b, bufg2_b, acc, accg2,
          sem_a, sem_b, ssem_a, ssem_b):
        c = lax.axis_index("c")
        s = lax.axis_index("s")
        base = s * _EPT
        col0 = c * 128
        slots = ((idx_a, idxg2_a, buf_a, bufg2_a, sem_a, ssem_a),
                 (idx_b, idxg2_b, buf_b, bufg2_b, sem_b, ssem_b))

        def _srcs(j):
            e0 = base + j * _CHUNK
            return (idx_hbm.at[pl.ds(e0, _CHUNK)],
                    idxg2_hbm.at[pl.ds(c * _N_EDGES + e0, _CHUNK)],
                    vg2_hbm.at[pl.ds(e0, _CHUNK)],
                    v0_hbm.at[pl.ds(e0, _CHUNK)],
                    v1_hbm.at[pl.ds(e0, _CHUNK)])

        def _start(j, slot):
            idx_v, idxg2_v, buf, bufg2, sem, _ = slot
            si, sg, sv2, sv0, sv1 = _srcs(j)
            pltpu.async_copy(si, idx_v, sem)
            pltpu.async_copy(sg, idxg2_v, sem)
            pltpu.async_copy(sv2, bufg2, sem)

            @pl.when(c == 0)
            def _():
                pltpu.async_copy(sv0, buf, sem)

            @pl.when(c == 1)
            def _():
                pltpu.async_copy(sv1, buf, sem)

        def _finish(j, slot):
            idx_v, idxg2_v, buf, bufg2, sem, ssem = slot
            si, sg, sv2, sv0, sv1 = _srcs(j)
            pltpu.make_async_copy(si, idx_v, sem).wait()
            pltpu.make_async_copy(sg, idxg2_v, sem).wait()
            pltpu.make_async_copy(sv2, bufg2, sem).wait()
            pltpu.make_async_copy(sv0, buf, sem).wait()
            pltpu.async_copy(buf, acc.at[idx_v], ssem, add=True)
            pltpu.async_copy(bufg2, accg2.at[idxg2_v], ssem, add=True)

        def _drain_scatter(slot):
            idx_v, idxg2_v, buf, bufg2, sem, ssem = slot
            pltpu.make_async_copy(buf, acc.at[idx_v], ssem).wait()
            pltpu.make_async_copy(bufg2, accg2.at[idxg2_v], ssem).wait()
        @pl.when(s < _NS - 1)
        def _():
            pltpu.sync_copy(
                x_hbm.at[pl.ds(s * 640, 640), pl.ds(col0, 128)],
                acc.at[pl.ds(s * 640, 640)])

        @pl.when(s == _NS - 1)
        def _():
            pltpu.sync_copy(
                x_hbm.at[pl.ds(9600, 400), pl.ds(col0, 128)],
                acc.at[pl.ds(9600, 400)])
            pltpu.sync_copy(xg2_hbm.at[pl.ds(c * _GROWS, _GROWS)], accg2)

        plsc.subcore_barrier()

        _start(0, slots[0])
        _start(1, slots[1])

        def body(i, carry):
            j0 = 2 * i
            _finish(j0, slots[0])
            _finish(j0 + 1, slots[1])

            @pl.when(j0 + 2 < _NCHUNK)
            def _():
                _drain_scatter(slots[0])
                _start(j0 + 2, slots[0])

            @pl.when(j0 + 3 < _NCHUNK)
            def _():
                _drain_scatter(slots[1])
                _start(j0 + 3, slots[1])

            return carry

        lax.fori_loop(0, _NCHUNK // 2, body, 0)
        _drain_scatter(slots[0])
        _drain_scatter(slots[1])
        plsc.subcore_barrier()

        @pl.when(s < _NS - 1)
        def _():
            pltpu.sync_copy(acc.at[pl.ds(s * 640, 640)],
                            om_hbm.at[pl.ds(s * 640, 640), pl.ds(col0, 128)])

        @pl.when(s == _NS - 1)
        def _():
            pltpu.sync_copy(acc.at[pl.ds(9600, 400)],
                            om_hbm.at[pl.ds(9600, 400), pl.ds(col0, 128)])
            pltpu.sync_copy(accg2, og2_hbm.at[pl.ds(c * _GROWS, _GROWS)])

    return k(v0, v1, vg2, x2d, xg2, idx, idxg2)


@jax.jit
def kernel(x, x_edge, edge_index, wigner_inv, edge_envelope, node_offset,
           W1, b1, ln_scale, ln_bias, W2, b2):
    w81 = wigner_inv.reshape(_N_EDGES, _J * _J)
    env = edge_envelope.reshape(_N_EDGES, 1)
    r = jnp.arange(_J * _J)[:, None]
    col_j = jnp.arange(_V)[None, :] // _C
    S = jnp.stack([((r % _J == k) & (r // _J == col_j)).astype(jnp.bfloat16)
                   for k in range(3)])
    W2T = jnp.stack([jnp.tile(W2[:, k * _C:(k + 1) * _C], (1, _J))
                     for k in range(3)]).astype(jnp.bfloat16)
    b2T = jnp.stack([jnp.tile(b2[k * _C:(k + 1) * _C], _J)[None, :]
                     for k in range(3)])
    pi = jnp.arange(_C)[:, None]
    pcol = jnp.arange(128)[None, :]
    P = jnp.stack([(pcol == 32 * q + pi).astype(jnp.bfloat16)
                   for q in range(4)])
    tgt = (edge_index[1] - node_offset).astype(jnp.int32)
    mask4 = jnp.stack([(tgt % 4 == q) for q in range(4)],
                      axis=1).astype(jnp.float32)
    v0, v1, vg2 = _edge_values(x_edge, w81, env, mask4, W1,
                               b1.reshape(1, -1), ln_scale.reshape(1, -1),
                               ln_bias.reshape(1, -1), S, W2T, b2T, P)
    x2d = x.reshape(_N_NODES, _V)
    xg2p = x2d[:, 256:288].reshape(2, _HN // 4, 128)
    xg2 = jnp.zeros((2 * _GROWS, 128), jnp.float32)
    xg2 = xg2.at[0:_HN // 4].set(xg2p[0])
    xg2 = xg2.at[_GROWS:_GROWS + _HN // 4].set(xg2p[1])
    grow = jnp.where(tgt < _HN, tgt, tgt - _HN) // 4
    idxg2 = jnp.concatenate([
        jnp.where(tgt < _HN, grow, _HN // 4),
        jnp.where(tgt >= _HN, grow, _HN // 4),
    ])
    om, og2 = _sc_scatter(v0, v1, vg2, x2d, xg2, tgt, idxg2)
    outg2 = jnp.concatenate(
        [og2[0:_HN // 4], og2[_GROWS:_GROWS + _HN // 4]],
        axis=0).reshape(_N_NODES, 32)
    out2d = jnp.concatenate([om, outg2], axis=1)
    return out2d.reshape(_N_NODES, _J, _C)


# speedup vs baseline: 1.0596x; 1.0596x over previous
import functools

import jax
import jax.numpy as jnp
from jax import lax
from jax.experimental import pallas as pl
from jax.experimental.pallas import tpu as pltpu
from jax.experimental.pallas import tpu_sc as plsc

_J = 9
_C = 32
_V = _J * _C
_N_NODES = 10000
_N_EDGES = 160000
_RESCALE = 16.0

_BE = 800

_NS = 16
_EPT = _N_EDGES // _NS
_CHUNK = 40
_NCHUNK = _EPT // _CHUNK
_HN = _N_NODES // 2
_TRASH = _HN


def _edge_kernel(xe_ref, w81_ref, env_ref, mask4_ref, W1_ref, b1_ref,
                 lns_ref, lnb_ref, S_ref, W2T_ref, b2T_ref, P_ref,
                 v0_ref, v1_ref, vg2_ref):
    h = jnp.dot(xe_ref[...], W1_ref[...], preferred_element_type=jnp.float32)
    h = h + b1_ref[...]
    mean = jnp.mean(h, axis=-1, keepdims=True)
    var = jnp.mean((h - mean) * (h - mean), axis=-1, keepdims=True)
    h = (h - mean) * jax.lax.rsqrt(var + 1e-5) * lns_ref[...] + lnb_ref[...]
    h = h * jax.nn.sigmoid(h)
    hb = h.astype(jnp.bfloat16)
    wb = w81_ref[...].astype(jnp.bfloat16)
    env = env_ref[...] * (1.0 / _RESCALE)
    v = None
    for k in range(3):
        a = jnp.dot(wb, S_ref[k], preferred_element_type=jnp.float32)
        bm = jnp.dot(hb, W2T_ref[k], preferred_element_type=jnp.float32)
        bm = bm + b2T_ref[k]
        v = a * bm if v is None else v + a * bm
    v = v * env
    v0_ref[...] = v[:, 0:128]
    v1_ref[...] = v[:, 128:256]
    vg232 = v[:, 256:288]
    g = None
    for q in range(4):
        vq = (mask4_ref[:, q:q + 1] * vg232).astype(jnp.bfloat16)
        gq = jnp.dot(vq, P_ref[q], preferred_element_type=jnp.float32)
        g = gq if g is None else g + gq
    vg2_ref[...] = g


def _edge_values(x_edge, wigner81, envelope, mask4, W1, b1, ln_scale,
                 ln_bias, S, W2T, b2T, P):
    n_blocks = _N_EDGES // _BE
    full = lambda shape: pl.BlockSpec(shape, lambda i: (0,) * len(shape))
    return pl.pallas_call(
        _edge_kernel,
        grid=(n_blocks,),
        in_specs=[
            pl.BlockSpec((_BE, 128), lambda i: (i, 0)),
            pl.BlockSpec((_BE, 81), lambda i: (i, 0)),
            pl.BlockSpec((_BE, 1), lambda i: (i, 0)),
            pl.BlockSpec((_BE, 4), lambda i: (i, 0)),
            full((128, 64)),
            full((1, 64)),
            full((1, 64)),
            full((1, 64)),
            full((3, 81, _V)),
            full((3, 64, _V)),
            full((3, 1, _V)),
            full((4, 32, 128)),
        ],
        out_specs=[
            pl.BlockSpec((_BE, 128), lambda i: (i, 0)),
            pl.BlockSpec((_BE, 128), lambda i: (i, 0)),
            pl.BlockSpec((_BE, 128), lambda i: (i, 0)),
        ],
        out_shape=[
            jax.ShapeDtypeStruct((_N_EDGES, 128), jnp.float32),
            jax.ShapeDtypeStruct((_N_EDGES, 128), jnp.float32),
            jax.ShapeDtypeStruct((_N_EDGES, 128), jnp.float32),
        ],
    )(x_edge, wigner81, envelope, mask4, W1, b1, ln_scale, ln_bias,
      S, W2T, b2T, P)


_GROWS = 1256


def _sc_scatter(v0, v1, vg2, xm, xg2, idx, idxg2):
    mesh = plsc.VectorSubcoreMesh(core_axis_name="c", subcore_axis_name="s")

    @functools.partial(
        pl.kernel,
        mesh=mesh,
        out_type=[
            jax.ShapeDtypeStruct((2 * _N_NODES, 128), jnp.float32),
            jax.ShapeDtypeStruct((2 * _GROWS, 128), jnp.float32),
        ],
        scratch_types=[
            pltpu.VMEM((_CHUNK,), jnp.int32),
            pltpu.VMEM((_CHUNK,), jnp.int32),
            pltpu.VMEM((_CHUNK, 128), jnp.float32),
            pltpu.VMEM((_CHUNK, 128), jnp.float32),
            pltpu.VMEM((_CHUNK,), jnp.int32),
            pltpu.VMEM((_CHUNK,), jnp.int32),
            pltpu.VMEM((_CHUNK, 128), jnp.float32),
            pltpu.VMEM((_CHUNK, 128), jnp.float32),
            pltpu.VMEM_SHARED((_N_NODES, 128), jnp.float32),
            pltpu.VMEM_SHARED((_GROWS, 128), jnp.float32),
            pltpu.SemaphoreType.DMA,
            pltpu.SemaphoreType.DMA,
        ],
    )
    def k(v0_hbm, v1_hbm, vg2_hbm, xm_hbm, xg2_hbm, idx_hbm, idxg2_hbm,
          om_hbm, og2_hbm, idx_a, idxg2_a, buf_a, bufg2_a,
          idx_b, idxg2_b, buf_b, bufg2_b, acc, accg2, sem_a, sem_b):
        c = lax.axis_index("c")
        s = lax.axis_index("s")
        base = s * _EPT
        slots = ((idx_a, idxg2_a, buf_a, bufg2_a, sem_a),
                 (idx_b, idxg2_b, buf_b, bufg2_b, sem_b))

        def _srcs(j):
            e0 = base + j * _CHUNK
            return (idx_hbm.at[pl.ds(e0, _CHUNK)],
                    idxg2_hbm.at[pl.ds(c * _N_EDGES + e0, _CHUNK)],
                    vg2_hbm.at[pl.ds(e0, _CHUNK)],
                    v0_hbm.at[pl.ds(e0, _CHUNK)],
                    v1_hbm.at[pl.ds(e0, _CHUNK)])

        def _start(j, slot):
            idx_v, idxg2_v, buf, bufg2, sem = slot
            si, sg, sv2, sv0, sv1 = _srcs(j)
            pltpu.async_copy(si, idx_v, sem)
            pltpu.async_copy(sg, idxg2_v, sem)
            pltpu.async_copy(sv2, bufg2, sem)

            @pl.when(c == 0)
            def _():
                pltpu.async_copy(sv0, buf, sem)

            @pl.when(c == 1)
            def _():
                pltpu.async_copy(sv1, buf, sem)

        def _finish(j, slot):
            idx_v, idxg2_v, buf, bufg2, sem = slot
            si, sg, sv2, sv0, sv1 = _srcs(j)
            pltpu.make_async_copy(si, idx_v, sem).wait()
            pltpu.make_async_copy(sg, idxg2_v, sem).wait()
            pltpu.make_async_copy(sv2, bufg2, sem).wait()
            pltpu.make_async_copy(sv0, buf, sem).wait()
            pltpu.sync_copy(buf, acc.at[idx_v], add=True)
            pltpu.sync_copy(bufg2, accg2.at[idxg2_v], add=True)
        @pl.when(s < _NS - 1)
        def _():
            pltpu.sync_copy(xm_hbm.at[pl.ds(c * _N_NODES + s * 640, 640)],
                            acc.at[pl.ds(s * 640, 640)])

        @pl.when(s == _NS - 1)
        def _():
            pltpu.sync_copy(xm_hbm.at[pl.ds(c * _N_NODES + 9600, 400)],
                            acc.at[pl.ds(9600, 400)])
            pltpu.sync_copy(xg2_hbm.at[pl.ds(c * _GROWS, _GROWS)], accg2)

        plsc.subcore_barrier()

        _start(0, slots[0])
        _start(1, slots[1])

        def body(i, carry):
            j0 = 2 * i
            _finish(j0, slots[0])

            @pl.when(j0 + 2 < _NCHUNK)
            def _():
                _start(j0 + 2, slots[0])

            _finish(j0 + 1, slots[1])

            @pl.when(j0 + 3 < _NCHUNK)
            def _():
                _start(j0 + 3, slots[1])

            return carry

        lax.fori_loop(0, _NCHUNK // 2, body, 0)
        plsc.subcore_barrier()

        @pl.when(s < _NS - 1)
        def _():
            pltpu.sync_copy(acc.at[pl.ds(s * 640, 640)],
                            om_hbm.at[pl.ds(c * _N_NODES + s * 640, 640)])

        @pl.when(s == _NS - 1)
        def _():
            pltpu.sync_copy(acc.at[pl.ds(9600, 400)],
                            om_hbm.at[pl.ds(c * _N_NODES + 9600, 400)])
            pltpu.sync_copy(accg2, og2_hbm.at[pl.ds(c * _GROWS, _GROWS)])

    return k(v0, v1, vg2, xm, xg2, idx, idxg2)


@jax.jit
def kernel(x, x_edge, edge_index, wigner_inv, edge_envelope, node_offset,
           W1, b1, ln_scale, ln_bias, W2, b2):
    w81 = wigner_inv.reshape(_N_EDGES, _J * _J)
    env = edge_envelope.reshape(_N_EDGES, 1)
    r = jnp.arange(_J * _J)[:, None]
    col_j = jnp.arange(_V)[None, :] // _C
    S = jnp.stack([((r % _J == k) & (r // _J == col_j)).astype(jnp.bfloat16)
                   for k in range(3)])
    W2T = jnp.stack([jnp.tile(W2[:, k * _C:(k + 1) * _C], (1, _J))
                     for k in range(3)]).astype(jnp.bfloat16)
    b2T = jnp.stack([jnp.tile(b2[k * _C:(k + 1) * _C], _J)[None, :]
                     for k in range(3)])
    pi = jnp.arange(_C)[:, None]
    pcol = jnp.arange(128)[None, :]
    P = jnp.stack([(pcol == 32 * q + pi).astype(jnp.bfloat16)
                   for q in range(4)])
    tgt = (edge_index[1] - node_offset).astype(jnp.int32)
    mask4 = jnp.stack([(tgt % 4 == q) for q in range(4)],
                      axis=1).astype(jnp.float32)
    v0, v1, vg2 = _edge_values(x_edge, w81, env, mask4, W1,
                               b1.reshape(1, -1), ln_scale.reshape(1, -1),
                               ln_bias.reshape(1, -1), S, W2T, b2T, P)
    x2d = x.reshape(_N_NODES, _V)
    xm = jnp.concatenate([x2d[:, 0:128], x2d[:, 128:256]], axis=0)
    xg2p = x2d[:, 256:288].reshape(2, _HN // 4, 128)
    xg2 = jnp.zeros((2 * _GROWS, 128), jnp.float32)
    xg2 = xg2.at[0:_HN // 4].set(xg2p[0])
    xg2 = xg2.at[_GROWS:_GROWS + _HN // 4].set(xg2p[1])
    grow = jnp.where(tgt < _HN, tgt, tgt - _HN) // 4
    idxg2 = jnp.concatenate([
        jnp.where(tgt < _HN, grow, _HN // 4),
        jnp.where(tgt >= _HN, grow, _HN // 4),
    ])
    om, og2 = _sc_scatter(v0, v1, vg2, xm, xg2, tgt, idxg2)
    outg2 = jnp.concatenate(
        [og2[0:_HN // 4], og2[_GROWS:_GROWS + _HN // 4]],
        axis=0).reshape(_N_NODES, 32)
    out2d = jnp.concatenate(
        [om[0:_N_NODES], om[_N_NODES:2 * _N_NODES], outg2], axis=1)
    return out2d.reshape(_N_NODES, _J, _C)


# speedup vs baseline: 1.1614x; 1.0961x over previous
import functools

import jax
import jax.numpy as jnp
from jax import lax
from jax.experimental import pallas as pl
from jax.experimental.pallas import tpu as pltpu
from jax.experimental.pallas import tpu_sc as plsc

_J = 9
_C = 32
_V = _J * _C
_N_NODES = 10000
_N_EDGES = 160000
_RESCALE = 16.0

_BE = 1600

_NS = 16
_EPT = _N_EDGES // _NS
_CHUNK = 40
_NCHUNK = _EPT // _CHUNK
_HN = _N_NODES // 2
_TRASH = _HN


def _edge_kernel(xe_ref, w81_ref, env_ref, mask4_ref, W1_ref, b1_ref,
                 lns_ref, lnb_ref, S_ref, W2T_ref, b2T_ref, P_ref,
                 v0_ref, v1_ref, vg2_ref):
    h = jnp.dot(xe_ref[...], W1_ref[...], preferred_element_type=jnp.float32)
    h = h + b1_ref[...]
    mean = jnp.mean(h, axis=-1, keepdims=True)
    var = jnp.mean((h - mean) * (h - mean), axis=-1, keepdims=True)
    h = (h - mean) * jax.lax.rsqrt(var + 1e-5) * lns_ref[...] + lnb_ref[...]
    h = h * jax.nn.sigmoid(h)
    hb = h.astype(jnp.bfloat16)
    wb = w81_ref[...].astype(jnp.bfloat16)
    env = env_ref[...] * (1.0 / _RESCALE)
    v = None
    for k in range(3):
        a = jnp.dot(wb, S_ref[k], preferred_element_type=jnp.float32)
        bm = jnp.dot(hb, W2T_ref[k], preferred_element_type=jnp.float32)
        bm = bm + b2T_ref[k]
        v = a * bm if v is None else v + a * bm
    v = v * env
    v0_ref[...] = v[:, 0:128]
    v1_ref[...] = v[:, 128:256]
    vg232 = v[:, 256:288]
    g = None
    for q in range(4):
        vq = (mask4_ref[:, q:q + 1] * vg232).astype(jnp.bfloat16)
        gq = jnp.dot(vq, P_ref[q], preferred_element_type=jnp.float32)
        g = gq if g is None else g + gq
    vg2_ref[...] = g


def _edge_values(x_edge, wigner81, envelope, mask4, W1, b1, ln_scale,
                 ln_bias, S, W2T, b2T, P):
    n_blocks = _N_EDGES // _BE
    full = lambda shape: pl.BlockSpec(shape, lambda i: (0,) * len(shape))
    return pl.pallas_call(
        _edge_kernel,
        grid=(n_blocks,),
        in_specs=[
            pl.BlockSpec((_BE, 128), lambda i: (i, 0)),
            pl.BlockSpec((_BE, 81), lambda i: (i, 0)),
            pl.BlockSpec((_BE, 1), lambda i: (i, 0)),
            pl.BlockSpec((_BE, 4), lambda i: (i, 0)),
            full((128, 64)),
            full((1, 64)),
            full((1, 64)),
            full((1, 64)),
            full((3, 81, _V)),
            full((3, 64, _V)),
            full((3, 1, _V)),
            full((4, 32, 128)),
        ],
        out_specs=[
            pl.BlockSpec((_BE, 128), lambda i: (i, 0)),
            pl.BlockSpec((_BE, 128), lambda i: (i, 0)),
            pl.BlockSpec((_BE, 128), lambda i: (i, 0)),
        ],
        out_shape=[
            jax.ShapeDtypeStruct((_N_EDGES, 128), jnp.float32),
            jax.ShapeDtypeStruct((_N_EDGES, 128), jnp.float32),
            jax.ShapeDtypeStruct((_N_EDGES, 128), jnp.float32),
        ],
    )(x_edge, wigner81, envelope, mask4, W1, b1, ln_scale, ln_bias,
      S, W2T, b2T, P)


_GROWS = 1256


def _sc_scatter(v0, v1, vg2, xm, xg2, idx, idxg2):
    mesh = plsc.VectorSubcoreMesh(core_axis_name="c", subcore_axis_name="s")

    @functools.partial(
        pl.kernel,
        mesh=mesh,
        out_type=[
            jax.ShapeDtypeStruct((2 * _N_NODES, 128), jnp.float32),
            jax.ShapeDtypeStruct((2 * _GROWS, 128), jnp.float32),
        ],
        scratch_types=[
            pltpu.VMEM((_CHUNK,), jnp.int32),
            pltpu.VMEM((_CHUNK,), jnp.int32),
            pltpu.VMEM((_CHUNK, 128), jnp.float32),
            pltpu.VMEM((_CHUNK, 128), jnp.float32),
            pltpu.VMEM((_CHUNK,), jnp.int32),
            pltpu.VMEM((_CHUNK,), jnp.int32),
            pltpu.VMEM((_CHUNK, 128), jnp.float32),
            pltpu.VMEM((_CHUNK, 128), jnp.float32),
            pltpu.VMEM_SHARED((_N_NODES, 128), jnp.float32),
            pltpu.VMEM_SHARED((_GROWS, 128), jnp.float32),
            pltpu.SemaphoreType.DMA,
            pltpu.SemaphoreType.DMA,
        ],
    )
    def k(v0_hbm, v1_hbm, vg2_hbm, xm_hbm, xg2_hbm, idx_hbm, idxg2_hbm,
          om_hbm, og2_hbm, idx_a, idxg2_a, buf_a, bufg2_a,
          idx_b, idxg2_b, buf_b, bufg2_b, acc, accg2, sem_a, sem_b):
        c = lax.axis_index("c")
        s = lax.axis_index("s")
        base = s * _EPT
        slots = ((idx_a, idxg2_a, buf_a, bufg2_a, sem_a),
                 (idx_b, idxg2_b, buf_b, bufg2_b, sem_b))

        def _srcs(j):
            e0 = base + j * _CHUNK
            return (idx_hbm.at[pl.ds(e0, _CHUNK)],
                    idxg2_hbm.at[pl.ds(c * _N_EDGES + e0, _CHUNK)],
                    vg2_hbm.at[pl.ds(e0, _CHUNK)],
                    v0_hbm.at[pl.ds(e0, _CHUNK)],
                    v1_hbm.at[pl.ds(e0, _CHUNK)])

        def _start(j, slot):
            idx_v, idxg2_v, buf, bufg2, sem = slot
            si, sg, sv2, sv0, sv1 = _srcs(j)
            pltpu.async_copy(si, idx_v, sem)
            pltpu.async_copy(sg, idxg2_v, sem)
            pltpu.async_copy(sv2, bufg2, sem)

            @pl.when(c == 0)
            def _():
                pltpu.async_copy(sv0, buf, sem)

            @pl.when(c == 1)
            def _():
                pltpu.async_copy(sv1, buf, sem)

        def _finish(j, slot):
            idx_v, idxg2_v, buf, bufg2, sem = slot
            si, sg, sv2, sv0, sv1 = _srcs(j)
            pltpu.make_async_copy(si, idx_v, sem).wait()
            pltpu.make_async_copy(sg, idxg2_v, sem).wait()
            pltpu.make_async_copy(sv2, bufg2, sem).wait()
            pltpu.make_async_copy(sv0, buf, sem).wait()
            pltpu.sync_copy(buf, acc.at[idx_v], add=True)
            pltpu.sync_copy(bufg2, accg2.at[idxg2_v], add=True)
        @pl.when(s < _NS - 1)
        def _():
            pltpu.sync_copy(xm_hbm.at[pl.ds(c * _N_NODES + s * 640, 640)],
                            acc.at[pl.ds(s * 640, 640)])

        @pl.when(s == _NS - 1)
        def _():
            pltpu.sync_copy(xm_hbm.at[pl.ds(c * _N_NODES + 9600, 400)],
                            acc.at[pl.ds(9600, 400)])
            pltpu.sync_copy(xg2_hbm.at[pl.ds(c * _GROWS, _GROWS)], accg2)

        plsc.subcore_barrier()

        _start(0, slots[0])
        _start(1, slots[1])

        def body(i, carry):
            j0 = 2 * i
            _finish(j0, slots[0])

            @pl.when(j0 + 2 < _NCHUNK)
            def _():
                _start(j0 + 2, slots[0])

            _finish(j0 + 1, slots[1])

            @pl.when(j0 + 3 < _NCHUNK)
            def _():
                _start(j0 + 3, slots[1])

            return carry

        lax.fori_loop(0, _NCHUNK // 2, body, 0)
        plsc.subcore_barrier()

        @pl.when(s < _NS - 1)
        def _():
            pltpu.sync_copy(acc.at[pl.ds(s * 640, 640)],
                            om_hbm.at[pl.ds(c * _N_NODES + s * 640, 640)])

        @pl.when(s == _NS - 1)
        def _():
            pltpu.sync_copy(acc.at[pl.ds(9600, 400)],
                            om_hbm.at[pl.ds(c * _N_NODES + 9600, 400)])
            pltpu.sync_copy(accg2, og2_hbm.at[pl.ds(c * _GROWS, _GROWS)])

    return k(v0, v1, vg2, xm, xg2, idx, idxg2)


@jax.jit
def kernel(x, x_edge, edge_index, wigner_inv, edge_envelope, node_offset,
           W1, b1, ln_scale, ln_bias, W2, b2):
    w81 = wigner_inv.reshape(_N_EDGES, _J * _J)
    env = edge_envelope.reshape(_N_EDGES, 1)
    r = jnp.arange(_J * _J)[:, None]
    col_j = jnp.arange(_V)[None, :] // _C
    S = jnp.stack([((r % _J == k) & (r // _J == col_j)).astype(jnp.bfloat16)
                   for k in range(3)])
    W2T = jnp.stack([jnp.tile(W2[:, k * _C:(k + 1) * _C], (1, _J))
                     for k in range(3)]).astype(jnp.bfloat16)
    b2T = jnp.stack([jnp.tile(b2[k * _C:(k + 1) * _C], _J)[None, :]
                     for k in range(3)])
    pi = jnp.arange(_C)[:, None]
    pcol = jnp.arange(128)[None, :]
    P = jnp.stack([(pcol == 32 * q + pi).astype(jnp.bfloat16)
                   for q in range(4)])
    tgt = (edge_index[1] - node_offset).astype(jnp.int32)
    mask4 = jnp.stack([(tgt % 4 == q) for q in range(4)],
                      axis=1).astype(jnp.float32)
    v0, v1, vg2 = _edge_values(x_edge, w81, env, mask4, W1,
                               b1.reshape(1, -1), ln_scale.reshape(1, -1),
                               ln_bias.reshape(1, -1), S, W2T, b2T, P)
    x2d = x.reshape(_N_NODES, _V)
    xm = jnp.concatenate([x2d[:, 0:128], x2d[:, 128:256]], axis=0)
    xg2p = x2d[:, 256:288].reshape(2, _HN // 4, 128)
    xg2 = jnp.zeros((2 * _GROWS, 128), jnp.float32)
    xg2 = xg2.at[0:_HN // 4].set(xg2p[0])
    xg2 = xg2.at[_GROWS:_GROWS + _HN // 4].set(xg2p[1])
    grow = jnp.where(tgt < _HN, tgt, tgt - _HN) // 4
    idxg2 = jnp.concatenate([
        jnp.where(tgt < _HN, grow, _HN // 4),
        jnp.where(tgt >= _HN, grow, _HN // 4),
    ])
    om, og2 = _sc_scatter(v0, v1, vg2, xm, xg2, tgt, idxg2)
    outg2 = jnp.concatenate(
        [og2[0:_HN // 4], og2[_GROWS:_GROWS + _HN // 4]],
        axis=0).reshape(_N_NODES, 32)
    out2d = jnp.concatenate(
        [om[0:_N_NODES], om[_N_NODES:2 * _N_NODES], outg2], axis=1)
    return out2d.reshape(_N_NODES, _J, _C)


# speedup vs baseline: 1.1773x; 1.0137x over previous
import functools

import jax
import jax.numpy as jnp
from jax import lax
from jax.experimental import pallas as pl
from jax.experimental.pallas import tpu as pltpu
from jax.experimental.pallas import tpu_sc as plsc

_J = 9
_C = 32
_V = _J * _C
_N_NODES = 10000
_N_EDGES = 160000
_RESCALE = 16.0

_BE = 3200

_NS = 16
_EPT = _N_EDGES // _NS
_CHUNK = 40
_NCHUNK = _EPT // _CHUNK
_HN = _N_NODES // 2
_TRASH = _HN


def _edge_kernel(xe_ref, w81_ref, env_ref, mask4_ref, W1_ref, b1_ref,
                 lns_ref, lnb_ref, S_ref, W2T_ref, b2T_ref, P_ref,
                 v0_ref, v1_ref, vg2_ref):
    h = jnp.dot(xe_ref[...], W1_ref[...], preferred_element_type=jnp.float32)
    h = h + b1_ref[...]
    mean = jnp.mean(h, axis=-1, keepdims=True)
    var = jnp.mean((h - mean) * (h - mean), axis=-1, keepdims=True)
    h = (h - mean) * jax.lax.rsqrt(var + 1e-5) * lns_ref[...] + lnb_ref[...]
    h = h * jax.nn.sigmoid(h)
    hb = h.astype(jnp.bfloat16)
    wb = w81_ref[...].astype(jnp.bfloat16)
    env = env_ref[...] * (1.0 / _RESCALE)
    v = None
    for k in range(3):
        a = jnp.dot(wb, S_ref[k], preferred_element_type=jnp.float32)
        bm = jnp.dot(hb, W2T_ref[k], preferred_element_type=jnp.float32)
        bm = bm + b2T_ref[k]
        v = a * bm if v is None else v + a * bm
    v = v * env
    v0_ref[...] = v[:, 0:128]
    v1_ref[...] = v[:, 128:256]
    vg232 = v[:, 256:288]
    g = None
    for q in range(4):
        vq = (mask4_ref[:, q:q + 1] * vg232).astype(jnp.bfloat16)
        gq = jnp.dot(vq, P_ref[q], preferred_element_type=jnp.float32)
        g = gq if g is None else g + gq
    vg2_ref[...] = g


def _edge_values(x_edge, wigner81, envelope, mask4, W1, b1, ln_scale,
                 ln_bias, S, W2T, b2T, P):
    n_blocks = _N_EDGES // _BE
    full = lambda shape: pl.BlockSpec(shape, lambda i: (0,) * len(shape))
    return pl.pallas_call(
        _edge_kernel,
        grid=(n_blocks,),
        in_specs=[
            pl.BlockSpec((_BE, 128), lambda i: (i, 0)),
            pl.BlockSpec((_BE, 81), lambda i: (i, 0)),
            pl.BlockSpec((_BE, 1), lambda i: (i, 0)),
            pl.BlockSpec((_BE, 4), lambda i: (i, 0)),
            full((128, 64)),
            full((1, 64)),
            full((1, 64)),
            full((1, 64)),
            full((3, 81, _V)),
            full((3, 64, _V)),
            full((3, 1, _V)),
            full((4, 32, 128)),
        ],
        out_specs=[
            pl.BlockSpec((_BE, 128), lambda i: (i, 0)),
            pl.BlockSpec((_BE, 128), lambda i: (i, 0)),
            pl.BlockSpec((_BE, 128), lambda i: (i, 0)),
        ],
        out_shape=[
            jax.ShapeDtypeStruct((_N_EDGES, 128), jnp.float32),
            jax.ShapeDtypeStruct((_N_EDGES, 128), jnp.float32),
            jax.ShapeDtypeStruct((_N_EDGES, 128), jnp.float32),
        ],
    )(x_edge, wigner81, envelope, mask4, W1, b1, ln_scale, ln_bias,
      S, W2T, b2T, P)


_GROWS = 1256


def _sc_scatter(v0, v1, vg2, xm, xg2, idx, idxg2):
    mesh = plsc.VectorSubcoreMesh(core_axis_name="c", subcore_axis_name="s")

    @functools.partial(
        pl.kernel,
        mesh=mesh,
        out_type=[
            jax.ShapeDtypeStruct((2 * _N_NODES, 128), jnp.float32),
            jax.ShapeDtypeStruct((2 * _GROWS, 128), jnp.float32),
        ],
        scratch_types=[
            pltpu.VMEM((_CHUNK,), jnp.int32),
            pltpu.VMEM((_CHUNK,), jnp.int32),
            pltpu.VMEM((_CHUNK, 128), jnp.float32),
            pltpu.VMEM((_CHUNK, 128), jnp.float32),
            pltpu.VMEM((_CHUNK,), jnp.int32),
            pltpu.VMEM((_CHUNK,), jnp.int32),
            pltpu.VMEM((_CHUNK, 128), jnp.float32),
            pltpu.VMEM((_CHUNK, 128), jnp.float32),
            pltpu.VMEM_SHARED((_N_NODES, 128), jnp.float32),
            pltpu.VMEM_SHARED((_GROWS, 128), jnp.float32),
            pltpu.SemaphoreType.DMA,
            pltpu.SemaphoreType.DMA,
        ],
    )
    def k(v0_hbm, v1_hbm, vg2_hbm, xm_hbm, xg2_hbm, idx_hbm, idxg2_hbm,
          om_hbm, og2_hbm, idx_a, idxg2_a, buf_a, bufg2_a,
          idx_b, idxg2_b, buf_b, bufg2_b, acc, accg2, sem_a, sem_b):
        c = lax.axis_index("c")
        s = lax.axis_index("s")
        base = s * _EPT
        slots = ((idx_a, idxg2_a, buf_a, bufg2_a, sem_a),
                 (idx_b, idxg2_b, buf_b, bufg2_b, sem_b))

        def _srcs(j):
            e0 = base + j * _CHUNK
            return (idx_hbm.at[pl.ds(e0, _CHUNK)],
                    idxg2_hbm.at[pl.ds(c * _N_EDGES + e0, _CHUNK)],
                    vg2_hbm.at[pl.ds(e0, _CHUNK)],
                    v0_hbm.at[pl.ds(e0, _CHUNK)],
                    v1_hbm.at[pl.ds(e0, _CHUNK)])

        def _start(j, slot):
            idx_v, idxg2_v, buf, bufg2, sem = slot
            si, sg, sv2, sv0, sv1 = _srcs(j)
            pltpu.async_copy(si, idx_v, sem)
            pltpu.async_copy(sg, idxg2_v, sem)
            pltpu.async_copy(sv2, bufg2, sem)

            @pl.when(c == 0)
            def _():
                pltpu.async_copy(sv0, buf, sem)

            @pl.when(c == 1)
            def _():
                pltpu.async_copy(sv1, buf, sem)

        def _finish(j, slot):
            idx_v, idxg2_v, buf, bufg2, sem = slot
            si, sg, sv2, sv0, sv1 = _srcs(j)
            pltpu.make_async_copy(si, idx_v, sem).wait()
            pltpu.make_async_copy(sg, idxg2_v, sem).wait()
            pltpu.make_async_copy(sv2, bufg2, sem).wait()
            pltpu.make_async_copy(sv0, buf, sem).wait()
            pltpu.sync_copy(buf, acc.at[idx_v], add=True)
            pltpu.sync_copy(bufg2, accg2.at[idxg2_v], add=True)
        @pl.when(s < _NS - 1)
        def _():
            pltpu.sync_copy(xm_hbm.at[pl.ds(c * _N_NODES + s * 640, 640)],
                            acc.at[pl.ds(s * 640, 640)])

        @pl.when(s == _NS - 1)
        def _():
            pltpu.sync_copy(xm_hbm.at[pl.ds(c * _N_NODES + 9600, 400)],
                            acc.at[pl.ds(9600, 400)])
            pltpu.sync_copy(xg2_hbm.at[pl.ds(c * _GROWS, _GROWS)], accg2)

        plsc.subcore_barrier()

        _start(0, slots[0])
        _start(1, slots[1])

        def body(i, carry):
            j0 = 2 * i
            _finish(j0, slots[0])

            @pl.when(j0 + 2 < _NCHUNK)
            def _():
                _start(j0 + 2, slots[0])

            _finish(j0 + 1, slots[1])

            @pl.when(j0 + 3 < _NCHUNK)
            def _():
                _start(j0 + 3, slots[1])

            return carry

        lax.fori_loop(0, _NCHUNK // 2, body, 0)
        plsc.subcore_barrier()

        @pl.when(s < _NS - 1)
        def _():
            pltpu.sync_copy(acc.at[pl.ds(s * 640, 640)],
                            om_hbm.at[pl.ds(c * _N_NODES + s * 640, 640)])

        @pl.when(s == _NS - 1)
        def _():
            pltpu.sync_copy(acc.at[pl.ds(9600, 400)],
                            om_hbm.at[pl.ds(c * _N_NODES + 9600, 400)])
            pltpu.sync_copy(accg2, og2_hbm.at[pl.ds(c * _GROWS, _GROWS)])

    return k(v0, v1, vg2, xm, xg2, idx, idxg2)


@jax.jit
def kernel(x, x_edge, edge_index, wigner_inv, edge_envelope, node_offset,
           W1, b1, ln_scale, ln_bias, W2, b2):
    w81 = wigner_inv.reshape(_N_EDGES, _J * _J)
    env = edge_envelope.reshape(_N_EDGES, 1)
    r = jnp.arange(_J * _J)[:, None]
    col_j = jnp.arange(_V)[None, :] // _C
    S = jnp.stack([((r % _J == k) & (r // _J == col_j)).astype(jnp.bfloat16)
                   for k in range(3)])
    W2T = jnp.stack([jnp.tile(W2[:, k * _C:(k + 1) * _C], (1, _J))
                     for k in range(3)]).astype(jnp.bfloat16)
    b2T = jnp.stack([jnp.tile(b2[k * _C:(k + 1) * _C], _J)[None, :]
                     for k in range(3)])
    pi = jnp.arange(_C)[:, None]
    pcol = jnp.arange(128)[None, :]
    P = jnp.stack([(pcol == 32 * q + pi).astype(jnp.bfloat16)
                   for q in range(4)])
    tgt = (edge_index[1] - node_offset).astype(jnp.int32)
    mask4 = jnp.stack([(tgt % 4 == q) for q in range(4)],
                      axis=1).astype(jnp.float32)
    v0, v1, vg2 = _edge_values(x_edge, w81, env, mask4, W1,
                               b1.reshape(1, -1), ln_scale.reshape(1, -1),
                               ln_bias.reshape(1, -1), S, W2T, b2T, P)
    x2d = x.reshape(_N_NODES, _V)
    xm = jnp.concatenate([x2d[:, 0:128], x2d[:, 128:256]], axis=0)
    xg2p = x2d[:, 256:288].reshape(2, _HN // 4, 128)
    xg2 = jnp.zeros((2 * _GROWS, 128), jnp.float32)
    xg2 = xg2.at[0:_HN // 4].set(xg2p[0])
    xg2 = xg2.at[_GROWS:_GROWS + _HN // 4].set(xg2p[1])
    grow = jnp.where(tgt < _HN, tgt, tgt - _HN) // 4
    idxg2 = jnp.concatenate([
        jnp.where(tgt < _HN, grow, _HN // 4),
        jnp.where(tgt >= _HN, grow, _HN // 4),
    ])
    om, og2 = _sc_scatter(v0, v1, vg2, xm, xg2, tgt, idxg2)
    outg2 = jnp.concatenate(
        [og2[0:_HN // 4], og2[_GROWS:_GROWS + _HN // 4]],
        axis=0).reshape(_N_NODES, 32)
    out2d = jnp.concatenate(
        [om[0:_N_NODES], om[_N_NODES:2 * _N_NODES], outg2], axis=1)
    return out2d.reshape(_N_NODES, _J, _C)
